# baseline (speedup 1.0000x reference)
import functools

import jax
import jax.numpy as jnp
from jax import lax
from jax.experimental import pallas as pl
from jax.experimental.pallas import tpu as pltpu
from jax.experimental.pallas import tpu_sc as plsc

SCALE_REL = 10.0

N_NODES = 100000
N_PAD = 100096
N_EDGES = 1600000
HALF = 16
NSUB = 16
PER_TILE = N_EDGES // NSUB
CHUNK = 400
NCHUNK = PER_TILE // CHUNK
ROWS_PER_TILE = N_PAD // NSUB
ZFULL = ROWS_PER_TILE // CHUNK
TAIL = ROWS_PER_TILE % CHUNK


def _prelu(x, a):
    return jnp.where(x > 0, x, a * x)


def _sc_propagate2(u1, v1, e1, u2, v2, e2, a_vec):
    mesh = plsc.VectorSubcoreMesh(core_axis_name="c", subcore_axis_name="s")

    @functools.partial(
        pl.kernel,
        mesh=mesh,
        compiler_params=pltpu.CompilerParams(use_tc_tiling_on_sc=False),
        out_type=jax.ShapeDtypeStruct((2, N_PAD, 2, HALF), jnp.float32),
        scratch_types=[
            pltpu.VMEM((CHUNK,), jnp.int32),
            pltpu.VMEM((CHUNK,), jnp.int32),
            pltpu.VMEM((CHUNK,), jnp.int32),
            pltpu.VMEM((CHUNK,), jnp.int32),
            pltpu.VMEM((CHUNK,), jnp.int32),
            pltpu.VMEM((CHUNK,), jnp.int32),
            pltpu.VMEM((CHUNK,), jnp.int32),
            pltpu.VMEM((CHUNK,), jnp.int32),
            pltpu.VMEM((CHUNK,), jnp.int32),
            pltpu.VMEM((CHUNK,), jnp.int32),
            pltpu.VMEM((CHUNK, HALF), jnp.float32),
            pltpu.VMEM((CHUNK, HALF), jnp.float32),
            pltpu.VMEM((CHUNK, HALF), jnp.float32),
            pltpu.VMEM((CHUNK, HALF), jnp.float32),
            pltpu.VMEM((HALF,), jnp.float32),
            pltpu.VMEM_SHARED((N_PAD, HALF), jnp.float32),
        ] + [pltpu.SemaphoreType.DMA] * 10,
    )
    def k(u1_h, v1_h, e1_h, u2_h, v2_h, e2_h, a_h, out_h,
          si0, si1, di0, di1, sp0, sp1, dp0, dp1, dsc0, dsc1,
          u0, u1b, v0, v1b, a_v, acc_sh,
          s_si0, s_si1, s_di0, s_di1, s_gu0, s_gu1, s_gv0, s_gv1,
          s_sc0, s_sc1):
        si = [si0, si1]
        di = [di0, di1]
        sp = [sp0, sp1]
        dp = [dp0, dp1]
        dsc = [dsc0, dsc1]
        uu = [u0, u1b]
        vv = [v0, v1b]
        s_si = [s_si0, s_si1]
        s_di = [s_di0, s_di1]
        s_gu = [s_gu0, s_gu1]
        s_gv = [s_gv0, s_gv1]
        s_sc = [s_sc0, s_sc1]

        c = lax.axis_index("c")
        s = lax.axis_index("s")
        row0 = s * ROWS_PER_TILE

        pltpu.sync_copy(a_h, a_v)
        a = a_v[...]
        cvec = jnp.full((HALF,), c, jnp.int32)
        base = s * PER_TILE

        def zero_body(i, _):
            u0[pl.ds(i * HALF, HALF), :] = jnp.zeros(
                (HALF, HALF), jnp.float32)
            return 0
        lax.fori_loop(0, CHUNK // HALF, zero_body, 0)

        def off(j):
            jw = lax.select(j < NCHUNK, j, j - NCHUNK)
            return base + jw * CHUNK

        for ps, (u_h, v_h, e_h) in enumerate(
                [(u1_h, v1_h, e1_h), (u2_h, v2_h, e2_h)]):
            if ps == 1:
                lax.fori_loop(0, CHUNK // HALF, zero_body, 0)
            for j in range(ZFULL):
                pltpu.sync_copy(u0.at[pl.ds(0, CHUNK)],
                                acc_sh.at[pl.ds(row0 + j * CHUNK, CHUNK)])
            pltpu.sync_copy(u0.at[pl.ds(0, TAIL)],
                            acc_sh.at[pl.ds(row0 + ZFULL * CHUNK, TAIL)])
            plsc.subcore_barrier()

            def issue_idx(b, j):
                o = off(j)
                pltpu.async_copy(e_h.at[pl.ds(o, CHUNK)], si[b], s_si[b])
                pltpu.async_copy(
                    e_h.at[pl.ds(N_EDGES + o, CHUNK)], di[b], s_di[b])

            def wait_idx(b):
                pltpu.make_async_copy(
                    e_h.at[pl.ds(0, CHUNK)], si[b], s_si[b]).wait()
                pltpu.make_async_copy(
                    e_h.at[pl.ds(0, CHUNK)], di[b], s_di[b]).wait()

            def expand_idx(b):
                def body(i, _):
                    sl = pl.ds(i * HALF, HALF)
                    x = si[b][sl]
                    sp[b][sl] = x + x + cvec
                    y = di[b][sl]
                    dp[b][sl] = y + y + cvec
                    return 0
                lax.fori_loop(0, CHUNK // HALF, body, 0)

            def issue_gather(b):
                pltpu.async_copy(u_h.at[sp[b]], uu[b], s_gu[b])
                pltpu.async_copy(v_h.at[dp[b]], vv[b], s_gv[b])

            def wait_gather(b):
                pltpu.make_async_copy(u_h.at[sp[b]], uu[b], s_gu[b]).wait()
                pltpu.make_async_copy(v_h.at[dp[b]], vv[b], s_gv[b]).wait()

            def wait_scatter(b):
                pltpu.make_async_copy(
                    uu[b], acc_sh.at[dsc[b]], s_sc[b]).wait()

            def phase(j, p, first):
                q = 1 - p
                wait_gather(p)
                for i in range(CHUNK // HALF):
                    sl = pl.ds(i * HALF, HALF)
                    dsc[p][sl] = di[p][sl]
                issue_idx(p, j + 2)
                wait_idx(q)
                expand_idx(q)
                if not first:
                    wait_scatter(q)
                issue_gather(q)

                def edge_grp(i, _):
                    for k2 in range(8):
                        e = i * 8 + k2
                        m = uu[p][e, :] - vv[p][e, :]
                        uu[p][e, :] = (jnp.maximum(m, 0.0)
                                       + a * jnp.minimum(m, 0.0))
                    return 0
                lax.fori_loop(0, CHUNK // 8, edge_grp, 0)

                pltpu.async_copy(uu[p], acc_sh.at[dsc[p]], s_sc[p],
                                 add=True)

            issue_idx(0, 0)
            issue_idx(1, 1)
            wait_idx(0)
            expand_idx(0)
            issue_gather(0)
            phase(0, 0, True)
            phase(1, 1, False)

            def loop_body(i, _):
                phase(2 * i, 0, False)
                phase(2 * i + 1, 1, False)
                return 0
            lax.fori_loop(1, NCHUNK // 2, loop_body, 0)

            wait_scatter(1)
            wait_gather(0)
            wait_idx(1)
            plsc.subcore_barrier()

            for j in range(ZFULL):
                pltpu.sync_copy(
                    acc_sh.at[pl.ds(row0 + j * CHUNK, CHUNK)],
                    out_h.at[ps, pl.ds(row0 + j * CHUNK, CHUNK), c, :])
            pltpu.sync_copy(
                acc_sh.at[pl.ds(row0 + ZFULL * CHUNK, TAIL)],
                out_h.at[ps, pl.ds(row0 + ZFULL * CHUNK, TAIL), c, :])


    return k(u1, v1, e1, u2, v2, e2, a_vec)


def _propagate_pair(Ua, Va, ea, Ub, Vb, eb, a_me):
    a_vec = jnp.full((HALF,), a_me, jnp.float32)
    sums = _sc_propagate2(
        Ua.reshape(2 * N_PAD, HALF), Va.reshape(2 * N_PAD, HALF), ea,
        Ub.reshape(2 * N_PAD, HALF), Vb.reshape(2 * N_PAD, HALF), eb,
        a_vec)
    out = []
    for i in range(2):
        S = sums[i].reshape(N_PAD, 2 * HALF)
        cnt = jnp.maximum(S[:, 31], 1.0)
        out.append(S[:, :30] / cnt[:, None])
    return out


def _padrows(x):
    return jnp.pad(x, ((0, N_PAD - x.shape[0]), (0, 0)))


def _padcols(w, extra_col=None):
    out = jnp.pad(w, ((0, 0), (0, 2)))
    return out


def kernel(tr, mask, A_in_sta, A_in_src, A_src_in_sta, pos_loc, pos_src,
           W_init, b_init, W_me, b_me, W_l1t1, b_l1t1, W_l1t2, b_l1t2,
           W_l2t1_1, b_l2t1_1, W_l2t1_2, b_l2t1_2, W_l2t2_1, b_l2t2_1,
           W_l2t2_2, b_l2t2_2, a_init, a_me, a11, a12, a1, a21, a22, a2):
    NH = W_init.shape[1]
    W_x, W_e = W_me[:NH], W_me[NH:]
    W_xp = _padcols(W_x)
    W_ep = _padcols(W_e) / (1000.0 * SCALE_REL)
    b_u = jnp.concatenate([b_me, jnp.zeros((1,), jnp.float32),
                           jnp.ones((1,), jnp.float32)])

    g_sta = _padrows(pos_loc[A_src_in_sta[0]]) @ W_ep
    g_src = _padrows(pos_src[A_src_in_sta[1]]) @ W_ep

    e_sta = A_in_sta.astype(jnp.int32).reshape(2 * N_EDGES)
    e_src = A_in_src.astype(jnp.int32).reshape(2 * N_EDGES)

    trm = _padrows(jnp.concatenate([tr, mask], 1))
    maskp = _padrows(mask)
    h = _prelu(trm @ W_init + b_init, a_init)

    U1 = _prelu(h, a11) @ W_xp + g_sta + b_u
    U2 = _prelu(h, a12) @ W_xp + g_src + b_u
    P1, P2 = _propagate_pair(U1, g_sta, e_sta, U2, g_src, e_src, a_me)
    t1 = (h @ W_l1t1[:NH] + P1 @ W_l1t1[NH:2 * NH]
          + maskp @ W_l1t1[2 * NH:] + b_l1t1)
    t2 = (h @ W_l1t2[:NH] + P2 @ W_l1t2[NH:2 * NH]
          + maskp @ W_l1t2[2 * NH:] + b_l1t2)
    ha = _prelu(t1, a1)
    hb = _prelu(t2, a1)

    y1 = _prelu(ha @ W_l2t1_1[:NH] + hb @ W_l2t1_1[NH:] + b_l2t1_1, a21)
    y2 = _prelu(ha @ W_l2t2_1[:NH] + hb @ W_l2t2_1[NH:] + b_l2t2_1, a22)
    U3 = y1 @ W_xp + g_sta + b_u
    U4 = y2 @ W_xp + g_src + b_u
    P3, P4 = _propagate_pair(U3, g_sta, e_sta, U4, g_src, e_src, a_me)
    t1 = (ha @ W_l2t1_2[:NH] + hb @ W_l2t1_2[NH:2 * NH]
          + P3 @ W_l2t1_2[2 * NH:3 * NH]
          + maskp @ W_l2t1_2[3 * NH:] + b_l2t1_2)
    t2 = (ha @ W_l2t2_2[:NH] + hb @ W_l2t2_2[NH:2 * NH]
          + P4 @ W_l2t2_2[2 * NH:3 * NH]
          + maskp @ W_l2t2_2[3 * NH:] + b_l2t2_2)
    out = _prelu(jnp.concatenate([t1, t2], 1), a2)
    return out[:N_NODES]


# speedup vs baseline: 1.2453x; 1.2453x over previous
import functools

import jax
import jax.numpy as jnp
from jax import lax
from jax.experimental import pallas as pl
from jax.experimental.pallas import tpu as pltpu
from jax.experimental.pallas import tpu_sc as plsc

SCALE_REL = 10.0

N_NODES = 100000
N_PAD = 100096
N_EDGES = 1600000
HALF = 16
NSUB = 16
PER_TILE = N_EDGES // NSUB
CHUNK = 400
NCHUNK = PER_TILE // CHUNK
ROWS_PER_TILE = N_PAD // NSUB
ZFULL = ROWS_PER_TILE // CHUNK
TAIL = ROWS_PER_TILE % CHUNK


def _prelu(x, a):
    return jnp.where(x > 0, x, a * x)


def _sc_propagate(u_tab, v_tab, edges, a_vec):
    mesh = plsc.VectorSubcoreMesh(core_axis_name="c", subcore_axis_name="s")

    @functools.partial(
        pl.kernel,
        mesh=mesh,
        compiler_params=pltpu.CompilerParams(use_tc_tiling_on_sc=False),
        out_type=jax.ShapeDtypeStruct((N_PAD, 2, HALF), jnp.float32),
        scratch_types=[
            pltpu.VMEM((CHUNK,), jnp.int32),
            pltpu.VMEM((CHUNK,), jnp.int32),
            pltpu.VMEM((CHUNK,), jnp.int32),
            pltpu.VMEM((CHUNK,), jnp.int32),
            pltpu.VMEM((CHUNK,), jnp.int32),
            pltpu.VMEM((CHUNK,), jnp.int32),
            pltpu.VMEM((CHUNK,), jnp.int32),
            pltpu.VMEM((CHUNK,), jnp.int32),
            pltpu.VMEM((CHUNK,), jnp.int32),
            pltpu.VMEM((CHUNK,), jnp.int32),
            pltpu.VMEM((CHUNK, HALF), jnp.float32),
            pltpu.VMEM((CHUNK, HALF), jnp.float32),
            pltpu.VMEM((CHUNK, HALF), jnp.float32),
            pltpu.VMEM((CHUNK, HALF), jnp.float32),
            pltpu.VMEM((HALF,), jnp.float32),
            pltpu.VMEM_SHARED((N_PAD, HALF), jnp.float32),
        ] + [pltpu.SemaphoreType.DMA] * 10,
    )
    def k(u_h, v_h, edges_h, a_h, out_h,
          si0, si1, di0, di1, sp0, sp1, dp0, dp1, dsc0, dsc1,
          u0, u1, v0, v1, a_v, acc_sh,
          s_si0, s_si1, s_di0, s_di1, s_gu0, s_gu1, s_gv0, s_gv1,
          s_sc0, s_sc1):
        si = [si0, si1]
        di = [di0, di1]
        sp = [sp0, sp1]
        dp = [dp0, dp1]
        dsc = [dsc0, dsc1]
        uu = [u0, u1]
        vv = [v0, v1]
        s_si = [s_si0, s_si1]
        s_di = [s_di0, s_di1]
        s_gu = [s_gu0, s_gu1]
        s_gv = [s_gv0, s_gv1]
        s_sc = [s_sc0, s_sc1]

        c = lax.axis_index("c")
        s = lax.axis_index("s")
        row0 = s * ROWS_PER_TILE

        pltpu.sync_copy(a_h, a_v)
        a = a_v[...]
        cvec = jnp.full((HALF,), c, jnp.int32)

        def zero_body(i, _):
            u0[pl.ds(i * HALF, HALF), :] = jnp.zeros(
                (HALF, HALF), jnp.float32)
            return 0
        lax.fori_loop(0, CHUNK // HALF, zero_body, 0)
        for j in range(ZFULL):
            pltpu.sync_copy(u0.at[pl.ds(0, CHUNK)],
                            acc_sh.at[pl.ds(row0 + j * CHUNK, CHUNK)])
        pltpu.sync_copy(u0.at[pl.ds(0, TAIL)],
                        acc_sh.at[pl.ds(row0 + ZFULL * CHUNK, TAIL)])
        plsc.subcore_barrier()

        base = s * PER_TILE

        def off(j):
            jw = lax.select(j < NCHUNK, j, j - NCHUNK)
            return base + jw * CHUNK

        def issue_idx(b, j):
            o = off(j)
            pltpu.async_copy(edges_h.at[pl.ds(o, CHUNK)], si[b], s_si[b])
            pltpu.async_copy(
                edges_h.at[pl.ds(N_EDGES + o, CHUNK)], di[b], s_di[b])

        def wait_idx(b):
            pltpu.make_async_copy(
                edges_h.at[pl.ds(0, CHUNK)], si[b], s_si[b]).wait()
            pltpu.make_async_copy(
                edges_h.at[pl.ds(0, CHUNK)], di[b], s_di[b]).wait()

        def expand_idx(b):
            def body(i, _):
                sl = pl.ds(i * HALF, HALF)
                x = si[b][sl]
                sp[b][sl] = x + x + cvec
                y = di[b][sl]
                dp[b][sl] = y + y + cvec
                return 0
            lax.fori_loop(0, CHUNK // HALF, body, 0)

        def issue_gather(b):
            pltpu.async_copy(u_h.at[sp[b]], uu[b], s_gu[b])
            pltpu.async_copy(v_h.at[dp[b]], vv[b], s_gv[b])

        def wait_gather(b):
            pltpu.make_async_copy(u_h.at[sp[b]], uu[b], s_gu[b]).wait()
            pltpu.make_async_copy(v_h.at[dp[b]], vv[b], s_gv[b]).wait()

        def wait_scatter(b):
            pltpu.make_async_copy(
                uu[b], acc_sh.at[dsc[b]], s_sc[b]).wait()

        def phase(j, p, first):
            q = 1 - p
            wait_gather(p)
            for i in range(CHUNK // HALF):
                sl = pl.ds(i * HALF, HALF)
                dsc[p][sl] = di[p][sl]
            issue_idx(p, j + 2)
            wait_idx(q)
            expand_idx(q)
            if not first:
                wait_scatter(q)
            issue_gather(q)

            def edge_grp(i, _):
                for k2 in range(8):
                    e = i * 8 + k2
                    m = uu[p][e, :] - vv[p][e, :]
                    uu[p][e, :] = (jnp.maximum(m, 0.0)
                                   + a * jnp.minimum(m, 0.0))
                return 0
            lax.fori_loop(0, CHUNK // 8, edge_grp, 0)

            pltpu.async_copy(uu[p], acc_sh.at[dsc[p]], s_sc[p], add=True)

        issue_idx(0, 0)
        issue_idx(1, 1)
        wait_idx(0)
        expand_idx(0)
        issue_gather(0)
        phase(0, 0, True)
        phase(1, 1, False)

        def loop_body(i, _):
            phase(2 * i, 0, False)
            phase(2 * i + 1, 1, False)
            return 0
        lax.fori_loop(1, NCHUNK // 2, loop_body, 0)

        wait_scatter(1)
        wait_gather(0)
        wait_idx(1)
        plsc.subcore_barrier()

        for j in range(ZFULL):
            pltpu.sync_copy(
                acc_sh.at[pl.ds(row0 + j * CHUNK, CHUNK)],
                out_h.at[pl.ds(row0 + j * CHUNK, CHUNK), c, :])
        pltpu.sync_copy(
            acc_sh.at[pl.ds(row0 + ZFULL * CHUNK, TAIL)],
            out_h.at[pl.ds(row0 + ZFULL * CHUNK, TAIL), c, :])

    return k(u_tab, v_tab, edges, a_vec)


def _propagate_raw(U32, V32, edges, a_me):
    u_tab = U32.reshape(2 * N_PAD, HALF)
    v_tab = V32.reshape(2 * N_PAD, HALF)
    a_vec = jnp.full((HALF,), a_me, jnp.float32)
    sums = _sc_propagate(u_tab, v_tab, edges, a_vec)
    return sums.reshape(N_PAD, 2 * HALF)


def _propagate_pair(Ua, Va, ea, Ub, Vb, eb, a_me):
    return (_propagate_raw(Ua, Va, ea, a_me),
            _propagate_raw(Ub, Vb, eb, a_me))


def _padrows(x):
    return jnp.pad(x, ((0, N_PAD - x.shape[0]), (0, 0)))


def _pc(w):
    return jnp.pad(w, ((0, 0), (0, 32 - w.shape[1])))


def _pr(w, rows):
    return jnp.pad(w, ((0, rows - w.shape[0]), (0, 0)))


BLK_AB = 4352
BLK_C = 4000


def _vspec(cols, blk=BLK_AB):
    return pl.BlockSpec((blk, cols), lambda i: (i, 0))


def _wspec(shape):
    return pl.BlockSpec(shape, lambda i: (0, 0))


def _stage_pre(trm, g_sta, g_src, W_init_p, b_init_p, W_xp32, b_u, sc):
    def body(trm_r, gsta_r, gsrc_r, wi_r, bi_r, wx_r, bu_r, sc_r,
             h_r, u1_r, u2_r):
        a_init, a11, a12 = sc_r[0], sc_r[1], sc_r[2]
        h = trm_r[...] @ wi_r[...] + bi_r[...]
        h = jnp.where(h > 0, h, a_init * h)
        h_r[...] = h
        x1 = jnp.where(h > 0, h, a11 * h)
        u1_r[...] = x1 @ wx_r[...] + gsta_r[...] + bu_r[...]
        x2 = jnp.where(h > 0, h, a12 * h)
        u2_r[...] = x2 @ wx_r[...] + gsrc_r[...] + bu_r[...]

    return pl.pallas_call(
        body,
        grid=(N_PAD // BLK_AB,),
        in_specs=[_vspec(28), _vspec(32), _vspec(32),
                  _wspec((28, 32)), _wspec((1, 32)),
                  _wspec((32, 32)), _wspec((1, 32)),
                  pl.BlockSpec(memory_space=pltpu.SMEM)],
        out_specs=[_vspec(32), _vspec(32), _vspec(32)],
        out_shape=[jax.ShapeDtypeStruct((N_PAD, 32), jnp.float32)] * 3,
    )(trm, g_sta, g_src, W_init_p, b_init_p, W_xp32, b_u, sc)


def _stage_mid(S1, S2, h, maskp, g_sta, g_src,
               Wh1, Wp1, Wm1, b1, Wh2, Wp2, Wm2, b2,
               Wa1, Wb1, by1, Wa2, Wb2, by2, W_xp32, b_u, sc):
    def body(s1_r, s2_r, h_r, m_r, gsta_r, gsrc_r,
             wh1_r, wp1_r, wm1_r, b1_r, wh2_r, wp2_r, wm2_r, b2_r,
             wa1_r, wb1_r, by1_r, wa2_r, wb2_r, by2_r, wx_r, bu_r, sc_r,
             ha_r, hb_r, u3_r, u4_r):
        a1, a21, a22 = sc_r[0], sc_r[1], sc_r[2]
        S1 = s1_r[...]
        S2 = s2_r[...]
        P1 = S1 / jnp.maximum(S1[:, 31:32], 1.0)
        P2 = S2 / jnp.maximum(S2[:, 31:32], 1.0)
        h = h_r[...]
        m = m_r[...]
        t1 = h @ wh1_r[...] + P1 @ wp1_r[...] + m @ wm1_r[...] + b1_r[...]
        t2 = h @ wh2_r[...] + P2 @ wp2_r[...] + m @ wm2_r[...] + b2_r[...]
        ha = jnp.where(t1 > 0, t1, a1 * t1)
        hb = jnp.where(t2 > 0, t2, a1 * t2)
        ha_r[...] = ha
        hb_r[...] = hb
        y1 = ha @ wa1_r[...] + hb @ wb1_r[...] + by1_r[...]
        y1 = jnp.where(y1 > 0, y1, a21 * y1)
        y2 = ha @ wa2_r[...] + hb @ wb2_r[...] + by2_r[...]
        y2 = jnp.where(y2 > 0, y2, a22 * y2)
        u3_r[...] = y1 @ wx_r[...] + gsta_r[...] + bu_r[...]
        u4_r[...] = y2 @ wx_r[...] + gsrc_r[...] + bu_r[...]

    return pl.pallas_call(
        body,
        grid=(N_PAD // BLK_AB,),
        in_specs=[_vspec(32), _vspec(32), _vspec(32), _vspec(16),
                  _vspec(32), _vspec(32)]
                 + [_wspec((32, 32)), _wspec((32, 32)), _wspec((16, 32)),
                    _wspec((1, 32))] * 2
                 + [_wspec((32, 32)), _wspec((32, 32)), _wspec((1, 32))] * 2
                 + [_wspec((32, 32)), _wspec((1, 32)),
                    pl.BlockSpec(memory_space=pltpu.SMEM)],
        out_specs=[_vspec(32)] * 4,
        out_shape=[jax.ShapeDtypeStruct((N_PAD, 32), jnp.float32)] * 4,
    )(S1, S2, h, maskp, g_sta, g_src,
      Wh1, Wp1, Wm1, b1, Wh2, Wp2, Wm2, b2,
      Wa1, Wb1, by1, Wa2, Wb2, by2, W_xp32, b_u, sc)


def _stage_final(S3, S4, ha, hb, maskp,
                 Wh1, Wp1, Wm1, Wk1, b1, Wh2, Wp2, Wm2, Wk2, b2, sc):
    def body(s3_r, s4_r, ha_r, hb_r, m_r,
             wh1_r, wp1_r, wm1_r, wk1_r, b1_r,
             wh2_r, wp2_r, wm2_r, wk2_r, b2_r, sc_r,
             o_r):
        a2 = sc_r[0]
        S3 = s3_r[...]
        S4 = s4_r[...]
        P3 = S3 / jnp.maximum(S3[:, 31:32], 1.0)
        P4 = S4 / jnp.maximum(S4[:, 31:32], 1.0)
        ha = ha_r[...]
        hb = hb_r[...]
        m = m_r[...]
        t1 = (ha @ wh1_r[...] + hb @ wp1_r[...] + P3 @ wm1_r[...]
              + m @ wk1_r[...] + b1_r[...])
        t2 = (ha @ wh2_r[...] + hb @ wp2_r[...] + P4 @ wm2_r[...]
              + m @ wk2_r[...] + b2_r[...])
        t = jnp.concatenate([t1[:, :15], t2[:, :15]], axis=1)
        o_r[...] = jnp.where(t > 0, t, a2 * t)

    return pl.pallas_call(
        body,
        grid=(N_NODES // BLK_C,),
        in_specs=[_vspec(32, BLK_C), _vspec(32, BLK_C), _vspec(32, BLK_C),
                  _vspec(32, BLK_C), _vspec(16, BLK_C)]
                 + [_wspec((32, 32)), _wspec((32, 32)), _wspec((32, 32)),
                    _wspec((16, 32)), _wspec((1, 32))] * 2
                 + [pl.BlockSpec(memory_space=pltpu.SMEM)],
        out_specs=[pl.BlockSpec((BLK_C, 30), lambda i: (i, 0))],
        out_shape=[jax.ShapeDtypeStruct((N_NODES, 30), jnp.float32)],
    )(S3, S4, ha, hb, maskp,
      Wh1, Wp1, Wm1, Wk1, b1, Wh2, Wp2, Wm2, Wk2, b2, sc)[0]


def kernel(tr, mask, A_in_sta, A_in_src, A_src_in_sta, pos_loc, pos_src,
           W_init, b_init, W_me, b_me, W_l1t1, b_l1t1, W_l1t2, b_l1t2,
           W_l2t1_1, b_l2t1_1, W_l2t1_2, b_l2t1_2, W_l2t2_1, b_l2t2_1,
           W_l2t2_2, b_l2t2_2, a_init, a_me, a11, a12, a1, a21, a22, a2):
    NH = W_init.shape[1]
    f32 = jnp.float32
    W_x, W_e = W_me[:NH], W_me[NH:]
    W_xp32 = _pr(_pc(W_x), 32)
    W_ep = _pc(W_e) / (1000.0 * SCALE_REL)
    b_u = jnp.concatenate([b_me, jnp.zeros((1,), f32),
                           jnp.ones((1,), f32)]).reshape(1, 32)

    g_sta = _padrows(pos_loc[A_src_in_sta[0]]) @ W_ep
    g_src = _padrows(pos_src[A_src_in_sta[1]]) @ W_ep

    e_sta = A_in_sta.astype(jnp.int32).reshape(2 * N_EDGES)
    e_src = A_in_src.astype(jnp.int32).reshape(2 * N_EDGES)

    trm = _padrows(jnp.concatenate([tr, mask], 1))
    maskp = jnp.pad(_padrows(mask), ((0, 0), (0, 6)))

    h, U1, U2 = _stage_pre(
        trm, g_sta, g_src, _pc(W_init), _pc(b_init.reshape(1, NH)),
        W_xp32, b_u, jnp.stack([a_init, a11, a12]))

    P = _propagate_pair(U1, g_sta, e_sta, U2, g_src, e_src, a_me)
    S1, S2 = P

    ha, hb, U3, U4 = _stage_mid(
        S1, S2, h, maskp, g_sta, g_src,
        _pr(_pc(W_l1t1[:NH]), 32), _pr(_pc(W_l1t1[NH:2 * NH]), 32),
        _pr(_pc(W_l1t1[2 * NH:]), 16), _pc(b_l1t1.reshape(1, NH)),
        _pr(_pc(W_l1t2[:NH]), 32), _pr(_pc(W_l1t2[NH:2 * NH]), 32),
        _pr(_pc(W_l1t2[2 * NH:]), 16), _pc(b_l1t2.reshape(1, NH)),
        _pr(_pc(W_l2t1_1[:NH]), 32), _pr(_pc(W_l2t1_1[NH:]), 32),
        _pc(b_l2t1_1.reshape(1, NH)),
        _pr(_pc(W_l2t2_1[:NH]), 32), _pr(_pc(W_l2t2_1[NH:]), 32),
        _pc(b_l2t2_1.reshape(1, NH)),
        W_xp32, b_u, jnp.stack([a1, a21, a22]))

    S3, S4 = _propagate_pair(U3, g_sta, e_sta, U4, g_src, e_src, a_me)

    return _stage_final(
        S3, S4, ha, hb, maskp,
        _pr(_pc(W_l2t1_2[:NH]), 32), _pr(_pc(W_l2t1_2[NH:2 * NH]), 32),
        _pr(_pc(W_l2t1_2[2 * NH:3 * NH]), 32),
        _pr(_pc(W_l2t1_2[3 * NH:]), 16),
        _pc(b_l2t1_2.reshape(1, 15)),
        _pr(_pc(W_l2t2_2[:NH]), 32), _pr(_pc(W_l2t2_2[NH:2 * NH]), 32),
        _pr(_pc(W_l2t2_2[2 * NH:3 * NH]), 32),
        _pr(_pc(W_l2t2_2[3 * NH:]), 16),
        _pc(b_l2t2_2.reshape(1, 15)),
        jnp.stack([a2]))


# speedup vs baseline: 1.3992x; 1.1236x over previous
import functools

import jax
import jax.numpy as jnp
from jax import lax
from jax.experimental import pallas as pl
from jax.experimental.pallas import tpu as pltpu
from jax.experimental.pallas import tpu_sc as plsc

SCALE_REL = 10.0

N_NODES = 100000
N_PAD = 100096
N_EDGES = 1600000
HALF = 16
NSUB = 16
PER_TILE = N_EDGES // NSUB
CHUNK = 400
NCHUNK = PER_TILE // CHUNK
ROWS_PER_TILE = N_PAD // NSUB
ZFULL = ROWS_PER_TILE // CHUNK
TAIL = ROWS_PER_TILE % CHUNK


def _prelu(x, a):
    return jnp.where(x > 0, x, a * x)


def _sc_propagate(u_tab, v_tab, edges, a_vec):
    mesh = plsc.VectorSubcoreMesh(core_axis_name="c", subcore_axis_name="s")

    @functools.partial(
        pl.kernel,
        mesh=mesh,
        compiler_params=pltpu.CompilerParams(use_tc_tiling_on_sc=False),
        out_type=jax.ShapeDtypeStruct((N_PAD, 2, HALF), jnp.float32),
        scratch_types=[
            pltpu.VMEM((CHUNK,), jnp.int32),
            pltpu.VMEM((CHUNK,), jnp.int32),
            pltpu.VMEM((CHUNK,), jnp.int32),
            pltpu.VMEM((CHUNK,), jnp.int32),
            pltpu.VMEM((CHUNK,), jnp.int32),
            pltpu.VMEM((CHUNK,), jnp.int32),
            pltpu.VMEM((CHUNK,), jnp.int32),
            pltpu.VMEM((CHUNK,), jnp.int32),
            pltpu.VMEM((CHUNK,), jnp.int32),
            pltpu.VMEM((CHUNK,), jnp.int32),
            pltpu.VMEM((CHUNK, HALF), jnp.float32),
            pltpu.VMEM((CHUNK, HALF), jnp.float32),
            pltpu.VMEM((CHUNK, HALF), jnp.float32),
            pltpu.VMEM((CHUNK, HALF), jnp.float32),
            pltpu.VMEM((HALF,), jnp.float32),
            pltpu.VMEM_SHARED((N_PAD, HALF), jnp.float32),
        ] + [pltpu.SemaphoreType.DMA] * 10,
    )
    def k(u_h, v_h, edges_h, a_h, out_h,
          si0, si1, di0, di1, sp0, sp1, dp0, dp1, dsc0, dsc1,
          u0, u1, v0, v1, a_v, acc_sh,
          s_si0, s_si1, s_di0, s_di1, s_gu0, s_gu1, s_gv0, s_gv1,
          s_sc0, s_sc1):
        si = [si0, si1]
        di = [di0, di1]
        sp = [sp0, sp1]
        dp = [dp0, dp1]
        dsc = [dsc0, dsc1]
        uu = [u0, u1]
        vv = [v0, v1]
        s_si = [s_si0, s_si1]
        s_di = [s_di0, s_di1]
        s_gu = [s_gu0, s_gu1]
        s_gv = [s_gv0, s_gv1]
        s_sc = [s_sc0, s_sc1]

        c = lax.axis_index("c")
        s = lax.axis_index("s")
        row0 = s * ROWS_PER_TILE

        pltpu.sync_copy(a_h, a_v)
        a = a_v[...]
        cvec = jnp.full((HALF,), c, jnp.int32)

        def zero_body(i, _):
            u0[pl.ds(i * HALF, HALF), :] = jnp.zeros(
                (HALF, HALF), jnp.float32)
            return 0
        lax.fori_loop(0, CHUNK // HALF, zero_body, 0)
        for j in range(ZFULL):
            pltpu.sync_copy(u0.at[pl.ds(0, CHUNK)],
                            acc_sh.at[pl.ds(row0 + j * CHUNK, CHUNK)])
        pltpu.sync_copy(u0.at[pl.ds(0, TAIL)],
                        acc_sh.at[pl.ds(row0 + ZFULL * CHUNK, TAIL)])
        plsc.subcore_barrier()

        base = s * PER_TILE

        def off(j):
            jw = lax.select(j < NCHUNK, j, j - NCHUNK)
            return base + jw * CHUNK

        def issue_idx(b, j):
            o = off(j)
            pltpu.async_copy(edges_h.at[pl.ds(o, CHUNK)], si[b], s_si[b])
            pltpu.async_copy(
                edges_h.at[pl.ds(N_EDGES + o, CHUNK)], di[b], s_di[b])

        def wait_idx(b):
            pltpu.make_async_copy(
                edges_h.at[pl.ds(0, CHUNK)], si[b], s_si[b]).wait()
            pltpu.make_async_copy(
                edges_h.at[pl.ds(0, CHUNK)], di[b], s_di[b]).wait()

        def expand_idx(b):
            def body(i, _):
                sl = pl.ds(i * HALF, HALF)
                x = si[b][sl]
                sp[b][sl] = x + x + cvec
                y = di[b][sl]
                dp[b][sl] = y + y + cvec
                return 0
            lax.fori_loop(0, CHUNK // HALF, body, 0)

        def issue_gather(b):
            pltpu.async_copy(u_h.at[sp[b]], uu[b], s_gu[b])
            pltpu.async_copy(v_h.at[dp[b]], vv[b], s_gv[b])

        def wait_gather(b):
            pltpu.make_async_copy(u_h.at[sp[b]], uu[b], s_gu[b]).wait()
            pltpu.make_async_copy(v_h.at[dp[b]], vv[b], s_gv[b]).wait()

        def wait_scatter(b):
            pltpu.make_async_copy(
                uu[b], acc_sh.at[dsc[b]], s_sc[b]).wait()

        def phase(j, p, first):
            q = 1 - p
            wait_gather(p)
            for i in range(CHUNK // HALF):
                sl = pl.ds(i * HALF, HALF)
                dsc[p][sl] = di[p][sl]
            issue_idx(p, j + 2)
            wait_idx(q)
            expand_idx(q)
            if not first:
                wait_scatter(q)
            issue_gather(q)

            def edge_grp(i, _):
                for k2 in range(8):
                    e = i * 8 + k2
                    m = uu[p][e, :] - vv[p][e, :]
                    uu[p][e, :] = (jnp.maximum(m, 0.0)
                                   + a * jnp.minimum(m, 0.0))
                return 0
            lax.fori_loop(0, CHUNK // 8, edge_grp, 0)

            pltpu.async_copy(uu[p], acc_sh.at[dsc[p]], s_sc[p], add=True)

        issue_idx(0, 0)
        issue_idx(1, 1)
        wait_idx(0)
        expand_idx(0)
        issue_gather(0)
        phase(0, 0, True)
        phase(1, 1, False)

        def loop_body(i, _):
            phase(2 * i, 0, False)
            phase(2 * i + 1, 1, False)
            return 0
        lax.fori_loop(1, NCHUNK // 2, loop_body, 0)

        wait_scatter(1)
        wait_gather(0)
        wait_idx(1)
        plsc.subcore_barrier()

        for j in range(ZFULL):
            pltpu.sync_copy(
                acc_sh.at[pl.ds(row0 + j * CHUNK, CHUNK)],
                out_h.at[pl.ds(row0 + j * CHUNK, CHUNK), c, :])
        pltpu.sync_copy(
            acc_sh.at[pl.ds(row0 + ZFULL * CHUNK, TAIL)],
            out_h.at[pl.ds(row0 + ZFULL * CHUNK, TAIL), c, :])

    return k(u_tab, v_tab, edges, a_vec)


def _propagate(U32, V32, edges, a_me):
    u_tab = U32.reshape(2 * N_PAD, HALF)
    v_tab = V32.reshape(2 * N_PAD, HALF)
    a_vec = jnp.full((HALF,), a_me, jnp.float32)
    sums = _sc_propagate(u_tab, v_tab, edges, a_vec)
    S = sums.reshape(N_PAD, 2 * HALF)
    cnt = jnp.maximum(S[:, 31], 1.0)
    return S[:, :30] / cnt[:, None]


def _padrows(x):
    return jnp.pad(x, ((0, N_PAD - x.shape[0]), (0, 0)))


def _padcols(w, extra_col=None):
    out = jnp.pad(w, ((0, 0), (0, 2)))
    return out


def kernel(tr, mask, A_in_sta, A_in_src, A_src_in_sta, pos_loc, pos_src,
           W_init, b_init, W_me, b_me, W_l1t1, b_l1t1, W_l1t2, b_l1t2,
           W_l2t1_1, b_l2t1_1, W_l2t1_2, b_l2t1_2, W_l2t2_1, b_l2t2_1,
           W_l2t2_2, b_l2t2_2, a_init, a_me, a11, a12, a1, a21, a22, a2):
    NH = W_init.shape[1]
    W_x, W_e = W_me[:NH], W_me[NH:]
    W_xp = _padcols(W_x)
    W_ep = _padcols(W_e) / (1000.0 * SCALE_REL)
    b_u = jnp.concatenate([b_me, jnp.zeros((1,), jnp.float32),
                           jnp.ones((1,), jnp.float32)])

    g_sta = _padrows(pos_loc[A_src_in_sta[0]]) @ W_ep
    g_src = _padrows(pos_src[A_src_in_sta[1]]) @ W_ep

    e_sta = A_in_sta.astype(jnp.int32).reshape(2 * N_EDGES)
    e_src = A_in_src.astype(jnp.int32).reshape(2 * N_EDGES)

    trm = _padrows(jnp.concatenate([tr, mask], 1))
    maskp = _padrows(mask)
    h = _prelu(trm @ W_init + b_init, a_init)

    U1 = _prelu(h, a11) @ W_xp + g_sta + b_u
    U2 = _prelu(h, a12) @ W_xp + g_src + b_u
    P1 = _propagate(U1, g_sta, e_sta, a_me)
    P2 = _propagate(U2, g_src, e_src, a_me)
    t1 = (h @ W_l1t1[:NH] + P1 @ W_l1t1[NH:2 * NH]
          + maskp @ W_l1t1[2 * NH:] + b_l1t1)
    t2 = (h @ W_l1t2[:NH] + P2 @ W_l1t2[NH:2 * NH]
          + maskp @ W_l1t2[2 * NH:] + b_l1t2)
    ha = _prelu(t1, a1)
    hb = _prelu(t2, a1)

    y1 = _prelu(ha @ W_l2t1_1[:NH] + hb @ W_l2t1_1[NH:] + b_l2t1_1, a21)
    y2 = _prelu(ha @ W_l2t2_1[:NH] + hb @ W_l2t2_1[NH:] + b_l2t2_1, a22)
    U3 = y1 @ W_xp + g_sta + b_u
    U4 = y2 @ W_xp + g_src + b_u
    P3 = _propagate(U3, g_sta, e_sta, a_me)
    P4 = _propagate(U4, g_src, e_src, a_me)
    t1 = (ha @ W_l2t1_2[:NH] + hb @ W_l2t1_2[NH:2 * NH]
          + P3 @ W_l2t1_2[2 * NH:3 * NH]
          + maskp @ W_l2t1_2[3 * NH:] + b_l2t1_2)
    t2 = (ha @ W_l2t2_2[:NH] + hb @ W_l2t2_2[NH:2 * NH]
          + P4 @ W_l2t2_2[2 * NH:3 * NH]
          + maskp @ W_l2t2_2[3 * NH:] + b_l2t2_2)
    out = _prelu(jnp.concatenate([t1, t2], 1), a2)
    return out[:N_NODES]
